# baseline (speedup 1.0000x reference)
import functools

import jax
import jax.numpy as jnp
from jax import lax
from jax.experimental import pallas as pl
from jax.experimental.pallas import tpu as pltpu
from jax.experimental.pallas import tpu_sc as plsc

B, T, H, D = 1024, 20, 26, 32
NC, NS = 2, 16
NW = NC * NS
RPW = B // NW
P = 32


def _tc_shift_body(ids_ref, offs_ref, out_ref):
    out_ref[...] = jnp.zeros((B, P, P), jnp.int32)
    out_ref[:, :T, :H] = ids_ref[...] + offs_ref[...]


_tc_shift = pl.pallas_call(
    _tc_shift_body,
    out_shape=jax.ShapeDtypeStruct((B, P, P), jnp.int32),
)


def _sc_body(ids_hbm, table_hbm, out_hbm, idx0_v, idx1_v, rows0_v, rows1_v,
             sem_g, sem_o):
    wid = lax.axis_index("s") * NC + lax.axis_index("c")

    def stage(b, idx_v):
        pltpu.sync_copy(ids_hbm.at[b], idx_v)

    def fire(idx_v, buf):
        for t in range(T):
            pltpu.async_copy(table_hbm.at[idx_v.at[t]], buf.at[t], sem_g)

    def drain(idx_v, buf):
        for t in range(T):
            pltpu.make_async_copy(table_hbm.at[idx_v.at[t]], buf.at[t],
                                  sem_g).wait()

    def out_copy(b, buf):
        for t in range(T):
            pltpu.async_copy(buf.at[t, pl.ds(0, H)], out_hbm.at[b, t],
                             sem_o)

    def out_wait(b, buf):
        for t in range(T):
            pltpu.make_async_copy(buf.at[t, pl.ds(0, H)], out_hbm.at[b, t],
                                  sem_o).wait()

    b0 = wid * RPW
    stage(b0, idx0_v)
    fire(idx0_v, rows0_v)

    def pair(p, carry):
        b = b0 + p * 2
        stage(b + 1, idx1_v)
        fire(idx1_v, rows1_v)
        drain(idx0_v, rows0_v)
        out_copy(b, rows0_v)

        @pl.when(p + 1 < RPW // 2)
        def _():
            stage(b + 2, idx0_v)
            out_wait(b, rows0_v)
            fire(idx0_v, rows0_v)

        drain(idx1_v, rows1_v)
        out_copy(b + 1, rows1_v)

        @pl.when(p + 1 < RPW // 2)
        def _():
            out_wait(b + 1, rows1_v)

        return carry

    lax.fori_loop(0, RPW // 2, pair, 0)
    out_wait(b0 + RPW - 2, rows0_v)
    out_wait(b0 + RPW - 1, rows1_v)


@functools.partial(
    pl.kernel,
    out_type=jax.ShapeDtypeStruct((B, T, H, D), jnp.float32),
    mesh=plsc.VectorSubcoreMesh(core_axis_name="c", subcore_axis_name="s"),
    scratch_types=[
        pltpu.VMEM((P, P), jnp.int32),
        pltpu.VMEM((P, P), jnp.int32),
        pltpu.VMEM((T, P, D), jnp.float32),
        pltpu.VMEM((T, P, D), jnp.float32),
        pltpu.SemaphoreType.DMA,
        pltpu.SemaphoreType.DMA,
    ],
    compiler_params=pltpu.CompilerParams(use_tc_tiling_on_sc=False),
)
def _sc_gather(ids_hbm, table_hbm, out_hbm, idx0_v, idx1_v, rows0_v,
               rows1_v, sem_g, sem_o):
    _sc_body(ids_hbm, table_hbm, out_hbm, idx0_v, idx1_v, rows0_v, rows1_v,
             sem_g, sem_o)


def kernel(input_ids, table, offsets):
    shifted = _tc_shift(input_ids, offsets.reshape(1, 1, H))
    return _sc_gather(shifted, table)


# speedup vs baseline: 1.7738x; 1.7738x over previous
import functools

import jax
import jax.numpy as jnp
from jax import lax
from jax.experimental import pallas as pl
from jax.experimental.pallas import tpu as pltpu
from jax.experimental.pallas import tpu_sc as plsc

B, T, H, D = 1024, 20, 26, 32
BTH = B * T * H
NC, NS = 2, 16
NW = NC * NS
PER_W = BTH // NW
CH = 128
NCH = PER_W // CH
K = 13
NG = NCH // K
VPC = CH // 16


def _sc_gather_kernel(ids_hbm, pat_hbm, table_hbm, out_hbm, idx_v, pat_v,
                      rows_v, sem):
    wid = lax.axis_index("s") * NC + lax.axis_index("c")
    base = wid * PER_W

    pltpu.sync_copy(ids_hbm.at[wid], idx_v)
    pltpu.sync_copy(pat_hbm.at[wid], pat_v)

    def group_body(g, carry):
        copies = []
        for k in range(K):
            c = g * K + k
            for j in range(VPC):
                sl = pl.ds(j * 16, 16)
                idx_v[c, sl] = idx_v[c, sl] + pat_v[c, sl]
            copies.append(
                pltpu.async_copy(table_hbm.at[idx_v.at[c]],
                                 rows_v.at[pl.ds(k * CH, CH)], sem))
        for cp in copies:
            cp.wait()
        pltpu.sync_copy(rows_v, out_hbm.at[pl.ds(base + g * (K * CH), K * CH)])
        return carry

    lax.fori_loop(0, NG, group_body, 0)


@functools.partial(
    pl.kernel,
    out_type=jax.ShapeDtypeStruct((BTH, D), jnp.float32),
    mesh=plsc.VectorSubcoreMesh(core_axis_name="c", subcore_axis_name="s"),
    scratch_types=[
        pltpu.VMEM((NCH, CH), jnp.int32),
        pltpu.VMEM((NCH, CH), jnp.int32),
        pltpu.VMEM((K * CH, D), jnp.float32),
        pltpu.SemaphoreType.DMA,
    ],
    compiler_params=pltpu.CompilerParams(use_tc_tiling_on_sc=False),
)
def _sc_gather(ids_hbm, pat_hbm, table_hbm, out_hbm, idx_v, pat_v, rows_v,
               sem):
    _sc_gather_kernel(ids_hbm, pat_hbm, table_hbm, out_hbm, idx_v, pat_v,
                      rows_v, sem)


def kernel(input_ids, table, offsets):
    ids = input_ids.reshape(NW, NCH, CH).astype(jnp.int32)
    pat = jnp.tile(offsets.astype(jnp.int32), BTH // H).reshape(NW, NCH, CH)
    out = _sc_gather(ids, pat, table)
    return out.reshape(B, T, H, D)
